# baseline (speedup 1.0000x reference)
import functools

import jax
import jax.numpy as jnp
from jax import lax
from jax.experimental import pallas as pl
from jax.experimental.pallas import tpu as pltpu
from jax.experimental.pallas import tpu_sc as plsc

_HI = lax.Precision.HIGHEST


def _edge_embed_body(ea_ref, w1, b1, w2, b2, w3, b3, e1_ref, e2_ref, e3_ref):
    ea = ea_ref[...]
    e1_ref[...] = jnp.dot(ea, w1[...], preferred_element_type=jnp.float32,
                          precision=_HI) + b1[...]
    e2_ref[...] = jnp.dot(ea, w2[...], preferred_element_type=jnp.float32,
                          precision=_HI) + b2[...]
    e3_ref[...] = jnp.dot(ea, w3[...], preferred_element_type=jnp.float32,
                          precision=_HI) + b3[...]


def _edge_embed(edge_attr, ws, bs):
    e_num, d_e = edge_attr.shape
    dins = [w.shape[1] for w in ws]
    be = 2000
    grid = e_num // be
    full = lambda i: (0, 0)
    return pl.pallas_call(
        _edge_embed_body,
        grid=(grid,),
        in_specs=[
            pl.BlockSpec((be, d_e), lambda i: (i, 0)),
            pl.BlockSpec((d_e, dins[0]), full), pl.BlockSpec((1, dins[0]), full),
            pl.BlockSpec((d_e, dins[1]), full), pl.BlockSpec((1, dins[1]), full),
            pl.BlockSpec((d_e, dins[2]), full), pl.BlockSpec((1, dins[2]), full),
        ],
        out_specs=[
            pl.BlockSpec((be, dins[0]), lambda i: (i, 0)),
            pl.BlockSpec((be, dins[1]), lambda i: (i, 0)),
            pl.BlockSpec((be, dins[2]), lambda i: (i, 0)),
        ],
        out_shape=[jax.ShapeDtypeStruct((e_num, d), jnp.float32) for d in dins],
    )(edge_attr, ws[0], bs[0][None, :], ws[1], bs[1][None, :], ws[2], bs[2][None, :])


_IB = 128
_GRP = 16
_ZC = 80
_NPAD = 16


def _zero_rows(buf, rows, din):
    def _zb(i, _):
        for s in range(din // 16):
            buf[i, pl.ds(s * 16, 16)] = jnp.zeros((16,), jnp.float32)
        return 0
    lax.fori_loop(0, rows, _zb, 0)


def _chunk_loop(sid, ns, n, body, zc=_ZC):
    nz_tot = n // zc
    nch = jnp.where(sid < (nz_tot % ns), nz_tot // ns + 1, nz_tot // ns)

    def _it(j, _):
        off = pl.multiple_of((j * ns + sid) * zc, 8)
        body(off)
        return 0
    lax.fori_loop(0, nch, _it, 0)


def _relu_add(hbuf, ebuf, din):
    def _ew(i, _):
        for r in range(2):
            ii = 2 * i + r
            for s in range(din // 16):
                sl = pl.ds(s * 16, 16)
                hbuf[ii, sl] = jnp.maximum(hbuf[ii, sl] + ebuf[ii, sl], 0.0)
        return 0
    lax.fori_loop(0, _IB // 2, _ew, 0)


@functools.cache
def _make_mp(n, ep, din, stage_h=False):
    info = plsc.get_sparse_core_info()
    nc, ns = info.num_cores, info.num_subcores
    nw = nc * ns
    nb = ep // _IB

    mesh = plsc.VectorSubcoreMesh(core_axis_name="c", subcore_axis_name="s")

    @functools.partial(
        pl.kernel,
        out_type=jax.ShapeDtypeStruct((nc, n, din), jnp.float32),
        mesh=mesh,
        scratch_types=[
            pltpu.VMEM_SHARED((n, din), jnp.float32),
            pltpu.VMEM((_IB,), jnp.int32),
            pltpu.VMEM((_IB,), jnp.int32),
            pltpu.VMEM((_IB, din), jnp.float32),
            pltpu.VMEM((_IB, din), jnp.float32),
            pltpu.SemaphoreType.DMA,
            pltpu.SemaphoreType.DMA,
        ],
    )
    def mp(h_hbm, ee_hbm, src_hbm, dst_hbm, out_hbm,
           aggr, srcv, dstv, hbuf, ebuf, gs, es):
        cid = lax.axis_index("c")
        sid = lax.axis_index("s")
        wid = cid * ns + sid
        lo = (wid * nb) // nw
        hi = ((wid + 1) * nb) // nw

        _zero_rows(ebuf, _ZC, din)
        _chunk_loop(sid, ns, n, lambda off: pltpu.sync_copy(
            ebuf.at[pl.ds(0, _ZC)], aggr.at[pl.ds(off, _ZC)]))

        plsc.subcore_barrier()

        def _batch(b, _):
            eb = pl.multiple_of(b * _IB, _IB)
            pltpu.sync_copy(src_hbm.at[pl.ds(eb, _IB)], srcv)
            pltpu.sync_copy(dst_hbm.at[pl.ds(eb, _IB)], dstv)
            pltpu.sync_copy(ee_hbm.at[pl.ds(eb, _IB)], ebuf)
            pltpu.async_copy(h_hbm.at[srcv], hbuf, gs).wait()
            _relu_add(hbuf, ebuf, din)
            pltpu.sync_copy(hbuf, aggr.at[dstv], add=True)
            return 0
        lax.fori_loop(lo, hi, _batch, 0)

        plsc.subcore_barrier()
        _chunk_loop(sid, ns, n, lambda off: pltpu.sync_copy(
            aggr.at[pl.ds(off, _ZC)], out_hbm.at[cid, pl.ds(off, _ZC)]))

    return mp


@functools.cache
def _make_mp_pair(n, ep):
    info = plsc.get_sparse_core_info()
    nc, ns = info.num_cores, info.num_subcores
    nw = nc * ns
    nb = ep // _IB
    nh = n // 2
    zc = 40

    mesh = plsc.VectorSubcoreMesh(core_axis_name="c", subcore_axis_name="s")

    @functools.partial(
        pl.kernel,
        out_type=jax.ShapeDtypeStruct((nc, nh, 128), jnp.float32),
        mesh=mesh,
        scratch_types=[
            pltpu.VMEM_SHARED((nh, 128), jnp.float32),
            pltpu.VMEM_SHARED((nh, 128), jnp.float32),
            pltpu.VMEM((_IB,), jnp.int32),
            pltpu.VMEM((_IB,), jnp.int32),
            pltpu.VMEM((_IB,), jnp.int32),
            pltpu.VMEM((_IB,), jnp.int32),
            pltpu.VMEM((_IB + 16,), jnp.int32),
            pltpu.VMEM((_IB + 16,), jnp.int32),
            pltpu.VMEM((_IB, 128), jnp.float32),
            pltpu.VMEM((_IB, 64), jnp.float32),
            pltpu.SemaphoreType.DMA,
            pltpu.SemaphoreType.DMA,
        ],
    )
    def mp(h_hbm, ee_hbm, src_hbm, dst_hbm, out_hbm,
           aggr, hspm, srcv, dstv, srch, dsth, spar, dpar, hbuf, ebuf, gs, es):
        cid = lax.axis_index("c")
        sid = lax.axis_index("s")
        wid = cid * ns + sid
        lo = (wid * nb) // nw
        hi = ((wid + 1) * nb) // nw

        def _stage(off):
            pltpu.sync_copy(h_hbm.at[pl.ds(off, zc)], hspm.at[pl.ds(off, zc)])
        _chunk_loop(sid, ns, nh, _stage, zc)
        _zero_rows(hbuf, zc, 128)
        _chunk_loop(sid, ns, nh, lambda off: pltpu.sync_copy(
            hbuf.at[pl.ds(0, zc)], aggr.at[pl.ds(off, zc)]), zc)
        plsc.subcore_barrier()

        def _batch(b, _):
            eb = pl.multiple_of(b * _IB, _IB)
            pltpu.sync_copy(src_hbm.at[pl.ds(eb, _IB)], srcv)
            pltpu.sync_copy(dst_hbm.at[pl.ds(eb, _IB)], dstv)

            def _idx(i, _):
                sl = pl.ds(i * 16, 16)
                sv = srcv[sl]
                dv = dstv[sl]
                srch[sl] = sv >> 1
                dsth[sl] = dv >> 1
                spar[sl] = (sv & 1) * 64
                dpar[sl] = (dv & 1) * 64
                return 0
            lax.fori_loop(0, _IB // 16, _idx, 0)

            pltpu.async_copy(ee_hbm.at[pl.ds(eb, _IB)], ebuf, es)
            pltpu.async_copy(hspm.at[srch], hbuf, gs).wait()
            pltpu.make_async_copy(ee_hbm.at[pl.ds(0, _IB)], ebuf, es).wait()

            zero16 = jnp.zeros((16,), jnp.float32)

            def _ew(i, _):
                sp_s = spar[pl.ds(i, 16)][0]
                dp_s = dpar[pl.ds(i, 16)][0]
                zp_s = 64 - dp_s
                for t in range(4):
                    hv = hbuf[i, pl.ds(sp_s + t * 16, 16)]
                    ev = ebuf[i, pl.ds(t * 16, 16)]
                    mv = jnp.maximum(hv + ev, 0.0)
                    hbuf[i, pl.ds(dp_s + t * 16, 16)] = mv
                    hbuf[i, pl.ds(zp_s + t * 16, 16)] = zero16
                return 0
            lax.fori_loop(0, _IB, _ew, 0)
            pltpu.sync_copy(hbuf, aggr.at[dsth], add=True)
            return 0
        lax.fori_loop(lo, hi, _batch, 0)

        plsc.subcore_barrier()
        _chunk_loop(sid, ns, nh, lambda off: pltpu.sync_copy(
            aggr.at[pl.ds(off, zc)], out_hbm.at[cid, pl.ds(off, zc)]), zc)

    return mp


def _node_body(h_ref, a_ref, w1, b1, w2, b2, o_ref):
    z = h_ref[...] + a_ref[0] + a_ref[1]
    t = jnp.maximum(jnp.dot(z, w1[...], preferred_element_type=jnp.float32,
                            precision=_HI) + b1[...], 0.0)
    o_ref[...] = jnp.maximum(jnp.dot(t, w2[...], preferred_element_type=jnp.float32,
                                     precision=_HI) + b2[...], 0.0)


def _node_update(h, aggr2, w1, b1, w2, b2, bn_rows=400):
    n, din = h.shape
    dm = w1.shape[1]
    dout = w2.shape[1]
    grid = n // bn_rows
    full = lambda i: (0, 0)
    return pl.pallas_call(
        _node_body,
        grid=(grid,),
        in_specs=[
            pl.BlockSpec((bn_rows, din), lambda i: (i, 0)),
            pl.BlockSpec((2, bn_rows, din), lambda i: (0, i, 0)),
            pl.BlockSpec((din, dm), full), pl.BlockSpec((1, dm), full),
            pl.BlockSpec((dm, dout), full), pl.BlockSpec((1, dout), full),
        ],
        out_specs=pl.BlockSpec((bn_rows, dout), lambda i: (i, 0)),
        out_shape=jax.ShapeDtypeStruct((n, dout), jnp.float32),
    )(h, aggr2, w1, b1[None, :], w2, b2[None, :])


def _node_pool_body(ng, h_ref, a_ref, batch_ref, w1, b1, w2, b2, o_ref, cnt):
    i = pl.program_id(0)

    @pl.when(i == 0)
    def _():
        o_ref[...] = jnp.zeros_like(o_ref)
        cnt[...] = jnp.zeros_like(cnt)

    z = h_ref[...] + a_ref[0] + a_ref[1]
    t = jnp.maximum(jnp.dot(z, w1[...], preferred_element_type=jnp.float32,
                            precision=_HI) + b1[...], 0.0)
    h3 = jnp.maximum(jnp.dot(t, w2[...], preferred_element_type=jnp.float32,
                             precision=_HI) + b2[...], 0.0)
    g = o_ref.shape[0]
    gids = lax.broadcasted_iota(jnp.int32, (g, h3.shape[0]), 0)
    onehot = (gids == batch_ref[0]).astype(jnp.float32)
    o_ref[...] += jnp.dot(onehot, h3, preferred_element_type=jnp.float32,
                          precision=_HI)
    cnt[...] += jnp.sum(onehot, axis=1, keepdims=True)

    @pl.when(i == ng - 1)
    def _():
        o_ref[...] = o_ref[...] / jnp.maximum(cnt[:, :1], 1.0)


def _node_update_pool(h, aggr2, batch3d, num_graphs, w1, b1, w2, b2, bn_rows=400):
    n, din = h.shape
    dm = w1.shape[1]
    dout = w2.shape[1]
    grid = n // bn_rows
    full = lambda i: (0, 0)
    return pl.pallas_call(
        functools.partial(_node_pool_body, grid),
        grid=(grid,),
        in_specs=[
            pl.BlockSpec((bn_rows, din), lambda i: (i, 0)),
            pl.BlockSpec((2, bn_rows, din), lambda i: (0, i, 0)),
            pl.BlockSpec((1, 1, bn_rows), lambda i: (i, 0, 0)),
            pl.BlockSpec((din, dm), full), pl.BlockSpec((1, dm), full),
            pl.BlockSpec((dm, dout), full), pl.BlockSpec((1, dout), full),
        ],
        out_specs=pl.BlockSpec((num_graphs, dout), lambda i: (0, 0)),
        out_shape=jax.ShapeDtypeStruct((num_graphs, dout), jnp.float32),
        scratch_shapes=[pltpu.VMEM((num_graphs, 128), jnp.float32)],
        compiler_params=pltpu.CompilerParams(
            dimension_semantics=("arbitrary",)),
    )(h, aggr2, batch3d, w1, b1[None, :], w2, b2[None, :])


def kernel(x, edge_index, edge_attr, batch, params, bn_stats):
    n, _ = x.shape
    e_num = edge_attr.shape[0]
    num_graphs = 64
    eps_bn = 1e-5

    fw2, fb2 = [], []
    for p, st in zip(params, bn_stats):
        scale = p["gamma"] / jnp.sqrt(st["var"] + eps_bn)
        fw2.append(p["W2"] * scale[None, :])
        fb2.append((p["b2"] - st["mean"]) * scale + p["beta"])

    ep = e_num
    src1d = edge_index[0]
    dst1d = edge_index[1]

    e1, e2, e3 = _edge_embed(edge_attr, [p["We"] for p in params],
                             [p["be"] for p in params])
    batch3d = batch.reshape(n // 400, 1, 400)

    h = x
    for li, ee in enumerate((e1, e2, e3)):
        p = params[li]
        if li == 0:
            aggr2 = _make_mp(n, ep, 128, False)(h, ee, src1d, dst1d)
        else:
            out_pair = _make_mp_pair(n, ep)(h.reshape(n // 2, 128), ee,
                                            src1d, dst1d)
            aggr2 = out_pair.reshape(2, n, 64)
        if li < 2:
            h = _node_update(h, aggr2, p["W1"], p["b1"], fw2[li], fb2[li])
        else:
            out = _node_update_pool(h, aggr2, batch3d, num_graphs,
                                    p["W1"], p["b1"], fw2[li], fb2[li])
    return out


# speedup vs baseline: 1.2454x; 1.2454x over previous
import functools

import jax
import jax.numpy as jnp
from jax import lax
from jax.experimental import pallas as pl
from jax.experimental.pallas import tpu as pltpu
from jax.experimental.pallas import tpu_sc as plsc

_HI = lax.Precision.HIGHEST


def _edge_embed_body(ea_ref, w1, b1, w2, b2, w3, b3, e1_ref, e2_ref, e3_ref):
    ea = ea_ref[...]
    e1_ref[...] = jnp.dot(ea, w1[...], preferred_element_type=jnp.float32,
                          precision=_HI) + b1[...]
    e2_ref[...] = jnp.dot(ea, w2[...], preferred_element_type=jnp.float32,
                          precision=_HI) + b2[...]
    e3_ref[...] = jnp.dot(ea, w3[...], preferred_element_type=jnp.float32,
                          precision=_HI) + b3[...]


def _edge_embed(edge_attr, ws, bs):
    e_num, d_e = edge_attr.shape
    dins = [w.shape[1] for w in ws]
    be = 2000
    grid = e_num // be
    full = lambda i: (0, 0)
    return pl.pallas_call(
        _edge_embed_body,
        grid=(grid,),
        in_specs=[
            pl.BlockSpec((be, d_e), lambda i: (i, 0)),
            pl.BlockSpec((d_e, dins[0]), full), pl.BlockSpec((1, dins[0]), full),
            pl.BlockSpec((d_e, dins[1]), full), pl.BlockSpec((1, dins[1]), full),
            pl.BlockSpec((d_e, dins[2]), full), pl.BlockSpec((1, dins[2]), full),
        ],
        out_specs=[
            pl.BlockSpec((be, dins[0]), lambda i: (i, 0)),
            pl.BlockSpec((be, dins[1]), lambda i: (i, 0)),
            pl.BlockSpec((be, dins[2]), lambda i: (i, 0)),
        ],
        out_shape=[jax.ShapeDtypeStruct((e_num, d), jnp.float32) for d in dins],
    )(edge_attr, ws[0], bs[0][None, :], ws[1], bs[1][None, :], ws[2], bs[2][None, :])


_IB = 128
_GRP = 16
_ZC = 80
_NPAD = 16


def _zero_rows(buf, rows, din):
    def _zb(i, _):
        for s in range(din // 16):
            buf[i, pl.ds(s * 16, 16)] = jnp.zeros((16,), jnp.float32)
        return 0
    lax.fori_loop(0, rows, _zb, 0)


def _chunk_loop(sid, ns, n, body):
    nz_tot = n // _ZC
    nch = jnp.where(sid < (nz_tot % ns), nz_tot // ns + 1, nz_tot // ns)

    def _it(j, _):
        off = pl.multiple_of((j * ns + sid) * _ZC, 8)
        body(off)
        return 0
    lax.fori_loop(0, nch, _it, 0)


def _relu_add(hbuf, ebuf, din):
    def _ew(i, _):
        for r in range(2):
            ii = 2 * i + r
            for s in range(din // 16):
                sl = pl.ds(s * 16, 16)
                hbuf[ii, sl] = jnp.maximum(hbuf[ii, sl] + ebuf[ii, sl], 0.0)
        return 0
    lax.fori_loop(0, _IB // 2, _ew, 0)


@functools.cache
def _make_mp(n, ep, din, stage_h=False):
    info = plsc.get_sparse_core_info()
    nc, ns = info.num_cores, info.num_subcores
    nw = nc * ns
    nb = ep // _IB

    mesh = plsc.VectorSubcoreMesh(core_axis_name="c", subcore_axis_name="s")

    @functools.partial(
        pl.kernel,
        out_type=jax.ShapeDtypeStruct((nc, n, din), jnp.float32),
        mesh=mesh,
        scratch_types=[
            pltpu.VMEM_SHARED((n, din), jnp.float32),
            pltpu.VMEM((_IB,), jnp.int32),
            pltpu.VMEM((_IB,), jnp.int32),
            pltpu.VMEM((_IB, din), jnp.float32),
            pltpu.VMEM((_IB, din), jnp.float32),
            pltpu.SemaphoreType.DMA,
            pltpu.SemaphoreType.DMA,
        ],
    )
    def mp(h_hbm, ee_hbm, src_hbm, dst_hbm, out_hbm,
           aggr, srcv, dstv, hbuf, ebuf, gs, es):
        cid = lax.axis_index("c")
        sid = lax.axis_index("s")
        wid = cid * ns + sid
        lo = (wid * nb) // nw
        hi = ((wid + 1) * nb) // nw

        _zero_rows(ebuf, _ZC, din)
        _chunk_loop(sid, ns, n, lambda off: pltpu.sync_copy(
            ebuf.at[pl.ds(0, _ZC)], aggr.at[pl.ds(off, _ZC)]))

        plsc.subcore_barrier()

        def _batch(b, _):
            eb = pl.multiple_of(b * _IB, _IB)
            pltpu.sync_copy(src_hbm.at[pl.ds(eb, _IB)], srcv)
            pltpu.sync_copy(dst_hbm.at[pl.ds(eb, _IB)], dstv)
            pltpu.sync_copy(ee_hbm.at[pl.ds(eb, _IB)], ebuf)
            pltpu.async_copy(h_hbm.at[srcv], hbuf, gs).wait()
            _relu_add(hbuf, ebuf, din)
            pltpu.sync_copy(hbuf, aggr.at[dstv], add=True)
            return 0
        lax.fori_loop(lo, hi, _batch, 0)

        plsc.subcore_barrier()
        _chunk_loop(sid, ns, n, lambda off: pltpu.sync_copy(
            aggr.at[pl.ds(off, _ZC)], out_hbm.at[cid, pl.ds(off, _ZC)]))

    return mp


def _node_body(h_ref, a_ref, w1, b1, w2, b2, o_ref):
    z = h_ref[...] + a_ref[0] + a_ref[1]
    t = jnp.maximum(jnp.dot(z, w1[...], preferred_element_type=jnp.float32,
                            precision=_HI) + b1[...], 0.0)
    o_ref[...] = jnp.maximum(jnp.dot(t, w2[...], preferred_element_type=jnp.float32,
                                     precision=_HI) + b2[...], 0.0)


def _node_update(h, aggr2, w1, b1, w2, b2, bn_rows=400):
    n, din = h.shape
    dm = w1.shape[1]
    dout = w2.shape[1]
    grid = n // bn_rows
    full = lambda i: (0, 0)
    return pl.pallas_call(
        _node_body,
        grid=(grid,),
        in_specs=[
            pl.BlockSpec((bn_rows, din), lambda i: (i, 0)),
            pl.BlockSpec((2, bn_rows, din), lambda i: (0, i, 0)),
            pl.BlockSpec((din, dm), full), pl.BlockSpec((1, dm), full),
            pl.BlockSpec((dm, dout), full), pl.BlockSpec((1, dout), full),
        ],
        out_specs=pl.BlockSpec((bn_rows, dout), lambda i: (i, 0)),
        out_shape=jax.ShapeDtypeStruct((n, dout), jnp.float32),
    )(h, aggr2, w1, b1[None, :], w2, b2[None, :])


def _node_pool_body(ng, h_ref, a_ref, batch_ref, w1, b1, w2, b2, o_ref, cnt):
    i = pl.program_id(0)

    @pl.when(i == 0)
    def _():
        o_ref[...] = jnp.zeros_like(o_ref)
        cnt[...] = jnp.zeros_like(cnt)

    z = h_ref[...] + a_ref[0] + a_ref[1]
    t = jnp.maximum(jnp.dot(z, w1[...], preferred_element_type=jnp.float32,
                            precision=_HI) + b1[...], 0.0)
    h3 = jnp.maximum(jnp.dot(t, w2[...], preferred_element_type=jnp.float32,
                             precision=_HI) + b2[...], 0.0)
    g = o_ref.shape[0]
    gids = lax.broadcasted_iota(jnp.int32, (g, h3.shape[0]), 0)
    onehot = (gids == batch_ref[0]).astype(jnp.float32)
    o_ref[...] += jnp.dot(onehot, h3, preferred_element_type=jnp.float32,
                          precision=_HI)
    cnt[...] += jnp.sum(onehot, axis=1, keepdims=True)

    @pl.when(i == ng - 1)
    def _():
        o_ref[...] = o_ref[...] / jnp.maximum(cnt[:, :1], 1.0)


def _node_update_pool(h, aggr2, batch3d, num_graphs, w1, b1, w2, b2, bn_rows=400):
    n, din = h.shape
    dm = w1.shape[1]
    dout = w2.shape[1]
    grid = n // bn_rows
    full = lambda i: (0, 0)
    return pl.pallas_call(
        functools.partial(_node_pool_body, grid),
        grid=(grid,),
        in_specs=[
            pl.BlockSpec((bn_rows, din), lambda i: (i, 0)),
            pl.BlockSpec((2, bn_rows, din), lambda i: (0, i, 0)),
            pl.BlockSpec((1, 1, bn_rows), lambda i: (i, 0, 0)),
            pl.BlockSpec((din, dm), full), pl.BlockSpec((1, dm), full),
            pl.BlockSpec((dm, dout), full), pl.BlockSpec((1, dout), full),
        ],
        out_specs=pl.BlockSpec((num_graphs, dout), lambda i: (0, 0)),
        out_shape=jax.ShapeDtypeStruct((num_graphs, dout), jnp.float32),
        scratch_shapes=[pltpu.VMEM((num_graphs, 128), jnp.float32)],
        compiler_params=pltpu.CompilerParams(
            dimension_semantics=("arbitrary",)),
    )(h, aggr2, batch3d, w1, b1[None, :], w2, b2[None, :])


def kernel(x, edge_index, edge_attr, batch, params, bn_stats):
    n, _ = x.shape
    e_num = edge_attr.shape[0]
    num_graphs = 64
    eps_bn = 1e-5

    dpad = 128
    wep, bep, w1p, b1s, fw2, fb2 = [], [], [], [], [], []
    for li, (p, st) in enumerate(zip(params, bn_stats)):
        din, dm = p["W1"].shape
        dout = p["W2"].shape[1]
        scale = p["gamma"] / jnp.sqrt(st["var"] + eps_bn)
        w2f = p["W2"] * scale[None, :]
        b2f = (p["b2"] - st["mean"]) * scale + p["beta"]
        wep.append(jnp.pad(p["We"], ((0, 0), (0, dpad - din))))
        bep.append(jnp.pad(p["be"], (0, dpad - din)))
        w1p.append(jnp.pad(p["W1"], ((0, dpad - din), (0, 0))))
        b1s.append(p["b1"])
        if li < 2:
            w2f = jnp.pad(w2f, ((0, 0), (0, dpad - dout)))
            b2f = jnp.pad(b2f, (0, dpad - dout))
        fw2.append(w2f)
        fb2.append(b2f)

    ep = e_num
    src1d = edge_index[0]
    dst1d = edge_index[1]

    e1, e2, e3 = _edge_embed(edge_attr, wep, bep)
    batch3d = batch.reshape(n // 400, 1, 400)

    h = x
    for li, ee in enumerate((e1, e2, e3)):
        mp = _make_mp(n, ep, 128, False)
        aggr2 = mp(h, ee, src1d, dst1d)
        if li < 2:
            h = _node_update(h, aggr2, w1p[li], b1s[li], fw2[li], fb2[li])
        else:
            out = _node_update_pool(h, aggr2, batch3d, num_graphs,
                                    w1p[li], b1s[li], fw2[li], fb2[li])
    return out


# speedup vs baseline: 1.3105x; 1.0523x over previous
import functools

import jax
import jax.numpy as jnp
from jax import lax
from jax.experimental import pallas as pl
from jax.experimental.pallas import tpu as pltpu
from jax.experimental.pallas import tpu_sc as plsc

_HI = lax.Precision.HIGHEST


def _edge_embed_body(ea_ref, w1, b1, w2, b2, w3, b3, e1_ref, e2_ref, e3_ref):
    ea = ea_ref[...]
    e1_ref[...] = jnp.dot(ea, w1[...], preferred_element_type=jnp.float32,
                          precision=_HI) + b1[...]
    e2_ref[...] = jnp.dot(ea, w2[...], preferred_element_type=jnp.float32,
                          precision=_HI) + b2[...]
    e3_ref[...] = jnp.dot(ea, w3[...], preferred_element_type=jnp.float32,
                          precision=_HI) + b3[...]


def _edge_embed(edge_attr, ws, bs):
    e_num, d_e = edge_attr.shape
    dins = [w.shape[1] for w in ws]
    be = 2000
    grid = e_num // be
    full = lambda i: (0, 0)
    return pl.pallas_call(
        _edge_embed_body,
        grid=(grid,),
        in_specs=[
            pl.BlockSpec((be, d_e), lambda i: (i, 0)),
            pl.BlockSpec((d_e, dins[0]), full), pl.BlockSpec((1, dins[0]), full),
            pl.BlockSpec((d_e, dins[1]), full), pl.BlockSpec((1, dins[1]), full),
            pl.BlockSpec((d_e, dins[2]), full), pl.BlockSpec((1, dins[2]), full),
        ],
        out_specs=[
            pl.BlockSpec((be, dins[0]), lambda i: (i, 0)),
            pl.BlockSpec((be, dins[1]), lambda i: (i, 0)),
            pl.BlockSpec((be, dins[2]), lambda i: (i, 0)),
        ],
        out_shape=[jax.ShapeDtypeStruct((e_num, d), jnp.float32) for d in dins],
    )(edge_attr, ws[0], bs[0][None, :], ws[1], bs[1][None, :], ws[2], bs[2][None, :])


_IB = 128
_GRP = 16
_ZC = 80
_NPAD = 16


def _zero_rows(buf, rows, din):
    def _zb(i, _):
        for s in range(din // 16):
            buf[i, pl.ds(s * 16, 16)] = jnp.zeros((16,), jnp.float32)
        return 0
    lax.fori_loop(0, rows, _zb, 0)


def _chunk_loop(sid, ns, n, body):
    nz_tot = n // _ZC
    nch = jnp.where(sid < (nz_tot % ns), nz_tot // ns + 1, nz_tot // ns)

    def _it(j, _):
        off = pl.multiple_of((j * ns + sid) * _ZC, 8)
        body(off)
        return 0
    lax.fori_loop(0, nch, _it, 0)


def _relu_add(hbuf, ebuf, din):
    def _ew(i, _):
        for r in range(2):
            ii = 2 * i + r
            for s in range(din // 16):
                sl = pl.ds(s * 16, 16)
                hbuf[ii, sl] = jnp.maximum(hbuf[ii, sl] + ebuf[ii, sl], 0.0)
        return 0
    lax.fori_loop(0, _IB // 2, _ew, 0)


@functools.cache
def _make_mp(n, ep, din, de=128):
    info = plsc.get_sparse_core_info()
    nc, ns = info.num_cores, info.num_subcores
    nw = nc * ns
    nb = ep // _IB

    mesh = plsc.VectorSubcoreMesh(core_axis_name="c", subcore_axis_name="s")

    @functools.partial(
        pl.kernel,
        out_type=jax.ShapeDtypeStruct((nc, n, din), jnp.float32),
        mesh=mesh,
        scratch_types=[
            pltpu.VMEM_SHARED((n, din), jnp.float32),
            pltpu.VMEM((_IB,), jnp.int32),
            pltpu.VMEM((_IB,), jnp.int32),
            pltpu.VMEM((_IB, din), jnp.float32),
            pltpu.VMEM((_IB, de), jnp.float32),
            pltpu.SemaphoreType.DMA,
            pltpu.SemaphoreType.DMA,
        ],
    )
    def mp(h_hbm, ee_hbm, src_hbm, dst_hbm, out_hbm,
           aggr, srcv, dstv, hbuf, ebuf, gs, es):
        cid = lax.axis_index("c")
        sid = lax.axis_index("s")
        wid = cid * ns + sid
        lo = (wid * nb) // nw
        hi = ((wid + 1) * nb) // nw

        _zero_rows(hbuf, _ZC, din)
        _chunk_loop(sid, ns, n, lambda off: pltpu.sync_copy(
            hbuf.at[pl.ds(0, _ZC)], aggr.at[pl.ds(off, _ZC)]))

        plsc.subcore_barrier()

        def _batch(b, _):
            eb = pl.multiple_of(b * _IB, _IB)
            pltpu.sync_copy(src_hbm.at[pl.ds(eb, _IB)], srcv)
            pltpu.sync_copy(dst_hbm.at[pl.ds(eb, _IB)], dstv)
            pltpu.sync_copy(ee_hbm.at[pl.ds(eb, _IB)], ebuf)
            pltpu.async_copy(h_hbm.at[srcv], hbuf, gs).wait()
            _relu_add(hbuf, ebuf, de)
            pltpu.sync_copy(hbuf, aggr.at[dstv], add=True)
            return 0
        lax.fori_loop(lo, hi, _batch, 0)

        plsc.subcore_barrier()
        _chunk_loop(sid, ns, n, lambda off: pltpu.sync_copy(
            aggr.at[pl.ds(off, _ZC)], out_hbm.at[cid, pl.ds(off, _ZC)]))

    return mp


def _node_body(h_ref, a_ref, w1, b1, w2, b2, o_ref):
    z = h_ref[...] + a_ref[0] + a_ref[1]
    t = jnp.maximum(jnp.dot(z, w1[...], preferred_element_type=jnp.float32,
                            precision=_HI) + b1[...], 0.0)
    o_ref[...] = jnp.maximum(jnp.dot(t, w2[...], preferred_element_type=jnp.float32,
                                     precision=_HI) + b2[...], 0.0)


def _node_update(h, aggr2, w1, b1, w2, b2, bn_rows=400):
    n, din = h.shape
    dm = w1.shape[1]
    dout = w2.shape[1]
    grid = n // bn_rows
    full = lambda i: (0, 0)
    return pl.pallas_call(
        _node_body,
        grid=(grid,),
        in_specs=[
            pl.BlockSpec((bn_rows, din), lambda i: (i, 0)),
            pl.BlockSpec((2, bn_rows, din), lambda i: (0, i, 0)),
            pl.BlockSpec((din, dm), full), pl.BlockSpec((1, dm), full),
            pl.BlockSpec((dm, dout), full), pl.BlockSpec((1, dout), full),
        ],
        out_specs=pl.BlockSpec((bn_rows, dout), lambda i: (i, 0)),
        out_shape=jax.ShapeDtypeStruct((n, dout), jnp.float32),
    )(h, aggr2, w1, b1[None, :], w2, b2[None, :])


def _node_pool_body(ng, h_ref, a_ref, batch_ref, w1, b1, w2, b2, o_ref, cnt):
    i = pl.program_id(0)

    @pl.when(i == 0)
    def _():
        o_ref[...] = jnp.zeros_like(o_ref)
        cnt[...] = jnp.zeros_like(cnt)

    z = h_ref[...] + a_ref[0] + a_ref[1]
    t = jnp.maximum(jnp.dot(z, w1[...], preferred_element_type=jnp.float32,
                            precision=_HI) + b1[...], 0.0)
    h3 = jnp.maximum(jnp.dot(t, w2[...], preferred_element_type=jnp.float32,
                             precision=_HI) + b2[...], 0.0)
    g = o_ref.shape[0]
    gids = lax.broadcasted_iota(jnp.int32, (g, h3.shape[0]), 0)
    onehot = (gids == batch_ref[0]).astype(jnp.float32)
    o_ref[...] += jnp.dot(onehot, h3, preferred_element_type=jnp.float32,
                          precision=_HI)
    cnt[...] += jnp.sum(onehot, axis=1, keepdims=True)

    @pl.when(i == ng - 1)
    def _():
        o_ref[...] = o_ref[...] / jnp.maximum(cnt[:, :1], 1.0)


def _node_update_pool(h, aggr2, batch3d, num_graphs, w1, b1, w2, b2, bn_rows=400):
    n, din = h.shape
    dm = w1.shape[1]
    dout = w2.shape[1]
    grid = n // bn_rows
    full = lambda i: (0, 0)
    return pl.pallas_call(
        functools.partial(_node_pool_body, grid),
        grid=(grid,),
        in_specs=[
            pl.BlockSpec((bn_rows, din), lambda i: (i, 0)),
            pl.BlockSpec((2, bn_rows, din), lambda i: (0, i, 0)),
            pl.BlockSpec((1, 1, bn_rows), lambda i: (i, 0, 0)),
            pl.BlockSpec((din, dm), full), pl.BlockSpec((1, dm), full),
            pl.BlockSpec((dm, dout), full), pl.BlockSpec((1, dout), full),
        ],
        out_specs=pl.BlockSpec((num_graphs, dout), lambda i: (0, 0)),
        out_shape=jax.ShapeDtypeStruct((num_graphs, dout), jnp.float32),
        scratch_shapes=[pltpu.VMEM((num_graphs, 128), jnp.float32)],
        compiler_params=pltpu.CompilerParams(
            dimension_semantics=("arbitrary",)),
    )(h, aggr2, batch3d, w1, b1[None, :], w2, b2[None, :])


def kernel(x, edge_index, edge_attr, batch, params, bn_stats):
    n, _ = x.shape
    e_num = edge_attr.shape[0]
    num_graphs = 64
    eps_bn = 1e-5

    dpad = 128
    w1p, b1s, fw2, fb2 = [], [], [], []
    for li, (p, st) in enumerate(zip(params, bn_stats)):
        din, dm = p["W1"].shape
        dout = p["W2"].shape[1]
        scale = p["gamma"] / jnp.sqrt(st["var"] + eps_bn)
        w2f = p["W2"] * scale[None, :]
        b2f = (p["b2"] - st["mean"]) * scale + p["beta"]
        w1p.append(jnp.pad(p["W1"], ((0, dpad - din), (0, 0))))
        b1s.append(p["b1"])
        if li < 2:
            w2f = jnp.pad(w2f, ((0, 0), (0, dpad - dout)))
            b2f = jnp.pad(b2f, (0, dpad - dout))
        fw2.append(w2f)
        fb2.append(b2f)

    ep = e_num
    src1d = edge_index[0]
    dst1d = edge_index[1]

    e1, e2, e3 = _edge_embed(edge_attr, [p["We"] for p in params],
                             [p["be"] for p in params])
    batch3d = batch.reshape(n // 400, 1, 400)

    h = x
    for li, ee in enumerate((e1, e2, e3)):
        mp = _make_mp(n, ep, 128, params[li]["We"].shape[1])
        aggr2 = mp(h, ee, src1d, dst1d)
        if li < 2:
            h = _node_update(h, aggr2, w1p[li], b1s[li], fw2[li], fb2[li])
        else:
            out = _node_update_pool(h, aggr2, batch3d, num_graphs,
                                    w1p[li], b1s[li], fw2[li], fb2[li])
    return out


# speedup vs baseline: 1.4311x; 1.0921x over previous
import functools

import jax
import jax.numpy as jnp
from jax import lax
from jax.experimental import pallas as pl
from jax.experimental.pallas import tpu as pltpu
from jax.experimental.pallas import tpu_sc as plsc

_HI = lax.Precision.HIGHEST


def _edge_embed_body(nl, ea_ref, *refs):
    ea = ea_ref[...]
    for j in range(nl):
        w, b = refs[2 * j], refs[2 * j + 1]
        refs[2 * nl + j][...] = jnp.dot(
            ea, w[...], preferred_element_type=jnp.float32,
            precision=_HI) + b[...]


def _edge_embed(edge_attr, ws, bs):
    e_num, d_e = edge_attr.shape
    dins = [w.shape[1] for w in ws]
    nl = len(ws)
    be = 2000
    grid = e_num // be
    full = lambda i: (0, 0)
    wspecs = []
    args = []
    for w, b in zip(ws, bs):
        wspecs += [pl.BlockSpec((d_e, w.shape[1]), full),
                   pl.BlockSpec((1, w.shape[1]), full)]
        args += [w, b[None, :]]
    out = pl.pallas_call(
        functools.partial(_edge_embed_body, nl),
        grid=(grid,),
        in_specs=[pl.BlockSpec((be, d_e), lambda i: (i, 0))] + wspecs,
        out_specs=[pl.BlockSpec((be, d), lambda i: (i, 0)) for d in dins],
        out_shape=[jax.ShapeDtypeStruct((e_num, d), jnp.float32) for d in dins],
    )(edge_attr, *args)
    return out


_IB = 128
_GRP = 16
_ZC = 80
_NPAD = 16


def _zero_rows(buf, rows, din):
    def _zb(i, _):
        for s in range(din // 16):
            buf[i, pl.ds(s * 16, 16)] = jnp.zeros((16,), jnp.float32)
        return 0
    lax.fori_loop(0, rows, _zb, 0)


def _chunk_loop(sid, ns, n, body):
    nz_tot = n // _ZC
    nch = jnp.where(sid < (nz_tot % ns), nz_tot // ns + 1, nz_tot // ns)

    def _it(j, _):
        off = pl.multiple_of((j * ns + sid) * _ZC, 8)
        body(off)
        return 0
    lax.fori_loop(0, nch, _it, 0)


def _relu_add(hbuf, ebuf, din):
    def _ew(i, _):
        for r in range(2):
            ii = 2 * i + r
            for s in range(din // 16):
                sl = pl.ds(s * 16, 16)
                hbuf[ii, sl] = jnp.maximum(hbuf[ii, sl] + ebuf[ii, sl], 0.0)
        return 0
    lax.fori_loop(0, _IB // 2, _ew, 0)


@functools.cache
def _make_mp(n, ep, din, de=128):
    info = plsc.get_sparse_core_info()
    nc, ns = info.num_cores, info.num_subcores
    nw = nc * ns
    nb = ep // _IB

    mesh = plsc.VectorSubcoreMesh(core_axis_name="c", subcore_axis_name="s")

    @functools.partial(
        pl.kernel,
        out_type=jax.ShapeDtypeStruct((nc, n, din), jnp.float32),
        mesh=mesh,
        scratch_types=[
            pltpu.VMEM_SHARED((n, din), jnp.float32),
            pltpu.VMEM((_IB,), jnp.int32),
            pltpu.VMEM((_IB,), jnp.int32),
            pltpu.VMEM((_IB, din), jnp.float32),
            pltpu.VMEM((_IB, de), jnp.float32),
            pltpu.SemaphoreType.DMA,
            pltpu.SemaphoreType.DMA,
        ],
    )
    def mp(h_hbm, ee_hbm, src_hbm, dst_hbm, out_hbm,
           aggr, srcv, dstv, hbuf, ebuf, gs, es):
        cid = lax.axis_index("c")
        sid = lax.axis_index("s")
        wid = cid * ns + sid
        lo = (wid * nb) // nw
        hi = ((wid + 1) * nb) // nw

        _zero_rows(hbuf, _ZC, din)
        _chunk_loop(sid, ns, n, lambda off: pltpu.sync_copy(
            hbuf.at[pl.ds(0, _ZC)], aggr.at[pl.ds(off, _ZC)]))

        plsc.subcore_barrier()

        def _batch(b, _):
            eb = pl.multiple_of(b * _IB, _IB)
            pltpu.sync_copy(src_hbm.at[pl.ds(eb, _IB)], srcv)
            pltpu.sync_copy(dst_hbm.at[pl.ds(eb, _IB)], dstv)
            pltpu.sync_copy(ee_hbm.at[pl.ds(eb, _IB)], ebuf)
            pltpu.async_copy(h_hbm.at[srcv], hbuf, gs).wait()
            _relu_add(hbuf, ebuf, de)
            pltpu.sync_copy(hbuf, aggr.at[dstv], add=True)
            return 0
        lax.fori_loop(lo, hi, _batch, 0)

        plsc.subcore_barrier()
        _chunk_loop(sid, ns, n, lambda off: pltpu.sync_copy(
            aggr.at[pl.ds(off, _ZC)], out_hbm.at[cid, pl.ds(off, _ZC)]))

    return mp


def _node_body(h_ref, a_ref, w1, b1, w2, b2, o_ref):
    z = h_ref[...] + a_ref[0] + a_ref[1]
    t = jnp.maximum(jnp.dot(z, w1[...], preferred_element_type=jnp.float32,
                            precision=_HI) + b1[...], 0.0)
    o_ref[...] = jnp.maximum(jnp.dot(t, w2[...], preferred_element_type=jnp.float32,
                                     precision=_HI) + b2[...], 0.0)


def _node_update(h, aggr2, w1, b1, w2, b2, bn_rows=400):
    n, din = h.shape
    dm = w1.shape[1]
    dout = w2.shape[1]
    grid = n // bn_rows
    full = lambda i: (0, 0)
    return pl.pallas_call(
        _node_body,
        grid=(grid,),
        in_specs=[
            pl.BlockSpec((bn_rows, din), lambda i: (i, 0)),
            pl.BlockSpec((2, bn_rows, din), lambda i: (0, i, 0)),
            pl.BlockSpec((din, dm), full), pl.BlockSpec((1, dm), full),
            pl.BlockSpec((dm, dout), full), pl.BlockSpec((1, dout), full),
        ],
        out_specs=pl.BlockSpec((bn_rows, dout), lambda i: (i, 0)),
        out_shape=jax.ShapeDtypeStruct((n, dout), jnp.float32),
    )(h, aggr2, w1, b1[None, :], w2, b2[None, :])


def _node_pool_body(ng, h_ref, a_ref, batch_ref, w1, b1, w2, b2, o_ref, cnt):
    i = pl.program_id(0)

    @pl.when(i == 0)
    def _():
        o_ref[...] = jnp.zeros_like(o_ref)
        cnt[...] = jnp.zeros_like(cnt)

    z = h_ref[...] + a_ref[0] + a_ref[1]
    t = jnp.maximum(jnp.dot(z, w1[...], preferred_element_type=jnp.float32,
                            precision=_HI) + b1[...], 0.0)
    h3 = jnp.maximum(jnp.dot(t, w2[...], preferred_element_type=jnp.float32,
                             precision=_HI) + b2[...], 0.0)
    g = o_ref.shape[0]
    gids = lax.broadcasted_iota(jnp.int32, (g, h3.shape[0]), 0)
    onehot = (gids == batch_ref[0]).astype(jnp.float32)
    o_ref[...] += jnp.dot(onehot, h3, preferred_element_type=jnp.float32,
                          precision=_HI)
    cnt[...] += jnp.sum(onehot, axis=1, keepdims=True)

    @pl.when(i == ng - 1)
    def _():
        o_ref[...] = o_ref[...] / jnp.maximum(cnt[:, :1], 1.0)


def _node_update_pool(h, aggr2, batch3d, num_graphs, w1, b1, w2, b2, bn_rows=400):
    n, din = h.shape
    dm = w1.shape[1]
    dout = w2.shape[1]
    grid = n // bn_rows
    full = lambda i: (0, 0)
    return pl.pallas_call(
        functools.partial(_node_pool_body, grid),
        grid=(grid,),
        in_specs=[
            pl.BlockSpec((bn_rows, din), lambda i: (i, 0)),
            pl.BlockSpec((2, bn_rows, din), lambda i: (0, i, 0)),
            pl.BlockSpec((1, 1, bn_rows), lambda i: (i, 0, 0)),
            pl.BlockSpec((din, dm), full), pl.BlockSpec((1, dm), full),
            pl.BlockSpec((dm, dout), full), pl.BlockSpec((1, dout), full),
        ],
        out_specs=pl.BlockSpec((num_graphs, dout), lambda i: (0, 0)),
        out_shape=jax.ShapeDtypeStruct((num_graphs, dout), jnp.float32),
        scratch_shapes=[pltpu.VMEM((num_graphs, 128), jnp.float32)],
        compiler_params=pltpu.CompilerParams(
            dimension_semantics=("arbitrary",)),
    )(h, aggr2, batch3d, w1, b1[None, :], w2, b2[None, :])


def kernel(x, edge_index, edge_attr, batch, params, bn_stats):
    n, _ = x.shape
    e_num = edge_attr.shape[0]
    num_graphs = 64
    eps_bn = 1e-5

    dpad = 128
    w1p, b1s, fw2, fb2 = [], [], [], []
    for li, (p, st) in enumerate(zip(params, bn_stats)):
        din, dm = p["W1"].shape
        dout = p["W2"].shape[1]
        scale = p["gamma"] / jnp.sqrt(st["var"] + eps_bn)
        w2f = p["W2"] * scale[None, :]
        b2f = (p["b2"] - st["mean"]) * scale + p["beta"]
        w1p.append(jnp.pad(p["W1"], ((0, dpad - din), (0, 0))))
        b1s.append(p["b1"])
        if li < 2:
            w2f = jnp.pad(w2f, ((0, 0), (0, dpad - dout)))
            b2f = jnp.pad(b2f, (0, dpad - dout))
        fw2.append(w2f)
        fb2.append(b2f)

    ep = e_num
    src1d = edge_index[0]
    dst1d = edge_index[1]

    [e1] = _edge_embed(edge_attr, [params[0]["We"]], [params[0]["be"]])
    batch3d = batch.reshape(n // 400, 1, 400)

    aggr2 = _make_mp(n, ep, 128, 128)(x, e1, src1d, dst1d)
    e2, e3 = _edge_embed(edge_attr, [p["We"] for p in params[1:]],
                         [p["be"] for p in params[1:]])
    h = _node_update(x, aggr2, w1p[0], b1s[0], fw2[0], fb2[0])
    for li, ee in ((1, e2), (2, e3)):
        mp = _make_mp(n, ep, 128, params[li]["We"].shape[1])
        aggr2 = mp(h, ee, src1d, dst1d)
        if li < 2:
            h = _node_update(h, aggr2, w1p[li], b1s[li], fw2[li], fb2[li])
        else:
            out = _node_update_pool(h, aggr2, batch3d, num_graphs,
                                    w1p[li], b1s[li], fw2[li], fb2[li])
    return out


# speedup vs baseline: 1.5298x; 1.0689x over previous
import functools

import jax
import jax.numpy as jnp
from jax import lax
from jax.experimental import pallas as pl
from jax.experimental.pallas import tpu as pltpu
from jax.experimental.pallas import tpu_sc as plsc

_HI = lax.Precision.HIGHEST


def _edge_embed_body(nl, ea_ref, *refs):
    ea = ea_ref[...]
    for j in range(nl):
        w, b = refs[2 * j], refs[2 * j + 1]
        refs[2 * nl + j][...] = jnp.dot(
            ea, w[...], preferred_element_type=jnp.float32,
            precision=_HI) + b[...]


def _edge_embed(edge_attr, ws, bs):
    e_num, d_e = edge_attr.shape
    dins = [w.shape[1] for w in ws]
    nl = len(ws)
    be = 2000
    grid = e_num // be
    full = lambda i: (0, 0)
    wspecs = []
    args = []
    for w, b in zip(ws, bs):
        wspecs += [pl.BlockSpec((d_e, w.shape[1]), full),
                   pl.BlockSpec((1, w.shape[1]), full)]
        args += [w, b[None, :]]
    out = pl.pallas_call(
        functools.partial(_edge_embed_body, nl),
        grid=(grid,),
        in_specs=[pl.BlockSpec((be, d_e), lambda i: (i, 0))] + wspecs,
        out_specs=[pl.BlockSpec((be, d), lambda i: (i, 0)) for d in dins],
        out_shape=[jax.ShapeDtypeStruct((e_num, d), jnp.float32) for d in dins],
    )(edge_attr, *args)
    return out


_IB = 128
_GRP = 16
_ZC = 80
_NPAD = 16


def _zero_rows(buf, rows, din):
    def _zb(i, _):
        for s in range(din // 16):
            buf[i, pl.ds(s * 16, 16)] = jnp.zeros((16,), jnp.float32)
        return 0
    lax.fori_loop(0, rows, _zb, 0)


def _chunk_loop(sid, ns, n, body):
    nz_tot = n // _ZC
    nch = jnp.where(sid < (nz_tot % ns), nz_tot // ns + 1, nz_tot // ns)

    def _it(j, _):
        off = pl.multiple_of((j * ns + sid) * _ZC, 8)
        body(off)
        return 0
    lax.fori_loop(0, nch, _it, 0)


def _relu_add(hbuf, ebuf, din):
    def _ew(i, _):
        for r in range(2):
            ii = 2 * i + r
            for s in range(din // 16):
                sl = pl.ds(s * 16, 16)
                hbuf[ii, sl] = jnp.maximum(hbuf[ii, sl] + ebuf[ii, sl], 0.0)
        return 0
    lax.fori_loop(0, _IB // 2, _ew, 0)


@functools.cache
def _make_mp(n, ep, din, de=128):
    info = plsc.get_sparse_core_info()
    nc, ns = info.num_cores, info.num_subcores
    nw = nc * ns
    nb = ep // _IB

    mesh = plsc.VectorSubcoreMesh(core_axis_name="c", subcore_axis_name="s")

    @functools.partial(
        pl.kernel,
        out_type=jax.ShapeDtypeStruct((nc, n, din), jnp.float32),
        mesh=mesh,
        scratch_types=[
            pltpu.VMEM_SHARED((n, din), jnp.float32),
            pltpu.VMEM((2, _IB), jnp.int32),
            pltpu.VMEM((_IB, din), jnp.float32),
            pltpu.VMEM((_IB, de), jnp.float32),
            pltpu.SemaphoreType.DMA,
            pltpu.SemaphoreType.DMA,
        ],
    )
    def mp(h_hbm, ee_hbm, idx_hbm, out_hbm, aggr, idxb, hbuf, ebuf, gs, es):
        cid = lax.axis_index("c")
        sid = lax.axis_index("s")
        wid = cid * ns + sid
        lo = (wid * nb) // nw
        hi = ((wid + 1) * nb) // nw

        _zero_rows(hbuf, _ZC, din)
        _chunk_loop(sid, ns, n, lambda off: pltpu.sync_copy(
            hbuf.at[pl.ds(0, _ZC)], aggr.at[pl.ds(off, _ZC)]))

        plsc.subcore_barrier()

        def _batch(b, _):
            eb = pl.multiple_of(b * _IB, _IB)
            pltpu.sync_copy(idx_hbm.at[b], idxb)
            pltpu.sync_copy(ee_hbm.at[pl.ds(eb, _IB)], ebuf)
            pltpu.async_copy(h_hbm.at[idxb.at[0]], hbuf, gs).wait()
            _relu_add(hbuf, ebuf, de)
            pltpu.sync_copy(hbuf, aggr.at[idxb.at[1]], add=True)
            return 0
        lax.fori_loop(lo, hi, _batch, 0)

        plsc.subcore_barrier()
        _chunk_loop(sid, ns, n, lambda off: pltpu.sync_copy(
            aggr.at[pl.ds(off, _ZC)], out_hbm.at[cid, pl.ds(off, _ZC)]))

    return mp


def _node_body(h_ref, a_ref, w1, b1, w2, b2, o_ref):
    z = h_ref[...] + a_ref[0] + a_ref[1]
    t = jnp.maximum(jnp.dot(z, w1[...], preferred_element_type=jnp.float32,
                            precision=_HI) + b1[...], 0.0)
    o_ref[...] = jnp.maximum(jnp.dot(t, w2[...], preferred_element_type=jnp.float32,
                                     precision=_HI) + b2[...], 0.0)


def _node_update(h, aggr2, w1, b1, w2, b2, bn_rows=400):
    n, din = h.shape
    dm = w1.shape[1]
    dout = w2.shape[1]
    grid = n // bn_rows
    full = lambda i: (0, 0)
    return pl.pallas_call(
        _node_body,
        grid=(grid,),
        in_specs=[
            pl.BlockSpec((bn_rows, din), lambda i: (i, 0)),
            pl.BlockSpec((2, bn_rows, din), lambda i: (0, i, 0)),
            pl.BlockSpec((din, dm), full), pl.BlockSpec((1, dm), full),
            pl.BlockSpec((dm, dout), full), pl.BlockSpec((1, dout), full),
        ],
        out_specs=pl.BlockSpec((bn_rows, dout), lambda i: (i, 0)),
        out_shape=jax.ShapeDtypeStruct((n, dout), jnp.float32),
    )(h, aggr2, w1, b1[None, :], w2, b2[None, :])


def _node_pool_body(ng, h_ref, a_ref, batch_ref, w1, b1, w2, b2, o_ref, cnt):
    i = pl.program_id(0)

    @pl.when(i == 0)
    def _():
        o_ref[...] = jnp.zeros_like(o_ref)
        cnt[...] = jnp.zeros_like(cnt)

    z = h_ref[...] + a_ref[0] + a_ref[1]
    t = jnp.maximum(jnp.dot(z, w1[...], preferred_element_type=jnp.float32,
                            precision=_HI) + b1[...], 0.0)
    h3 = jnp.maximum(jnp.dot(t, w2[...], preferred_element_type=jnp.float32,
                             precision=_HI) + b2[...], 0.0)
    g = o_ref.shape[0]
    gids = lax.broadcasted_iota(jnp.int32, (g, h3.shape[0]), 0)
    onehot = (gids == batch_ref[0]).astype(jnp.float32)
    o_ref[...] += jnp.dot(onehot, h3, preferred_element_type=jnp.float32,
                          precision=_HI)
    cnt[...] += jnp.sum(onehot, axis=1, keepdims=True)

    @pl.when(i == ng - 1)
    def _():
        o_ref[...] = o_ref[...] / jnp.maximum(cnt[:, :1], 1.0)


def _node_update_pool(h, aggr2, batch3d, num_graphs, w1, b1, w2, b2, bn_rows=400):
    n, din = h.shape
    dm = w1.shape[1]
    dout = w2.shape[1]
    grid = n // bn_rows
    full = lambda i: (0, 0)
    return pl.pallas_call(
        functools.partial(_node_pool_body, grid),
        grid=(grid,),
        in_specs=[
            pl.BlockSpec((bn_rows, din), lambda i: (i, 0)),
            pl.BlockSpec((2, bn_rows, din), lambda i: (0, i, 0)),
            pl.BlockSpec((1, 1, bn_rows), lambda i: (i, 0, 0)),
            pl.BlockSpec((din, dm), full), pl.BlockSpec((1, dm), full),
            pl.BlockSpec((dm, dout), full), pl.BlockSpec((1, dout), full),
        ],
        out_specs=pl.BlockSpec((num_graphs, dout), lambda i: (0, 0)),
        out_shape=jax.ShapeDtypeStruct((num_graphs, dout), jnp.float32),
        scratch_shapes=[pltpu.VMEM((num_graphs, 128), jnp.float32)],
        compiler_params=pltpu.CompilerParams(
            dimension_semantics=("arbitrary",)),
    )(h, aggr2, batch3d, w1, b1[None, :], w2, b2[None, :])


def kernel(x, edge_index, edge_attr, batch, params, bn_stats):
    n, _ = x.shape
    e_num = edge_attr.shape[0]
    num_graphs = 64
    eps_bn = 1e-5

    dpad = 128
    w1p, b1s, fw2, fb2 = [], [], [], []
    for li, (p, st) in enumerate(zip(params, bn_stats)):
        din, dm = p["W1"].shape
        dout = p["W2"].shape[1]
        scale = p["gamma"] / jnp.sqrt(st["var"] + eps_bn)
        w2f = p["W2"] * scale[None, :]
        b2f = (p["b2"] - st["mean"]) * scale + p["beta"]
        w1p.append(jnp.pad(p["W1"], ((0, dpad - din), (0, 0))))
        b1s.append(p["b1"])
        if li < 2:
            w2f = jnp.pad(w2f, ((0, 0), (0, dpad - dout)))
            b2f = jnp.pad(b2f, (0, dpad - dout))
        fw2.append(w2f)
        fb2.append(b2f)

    ep = e_num
    idx3d = jnp.stack((edge_index[0].reshape(ep // _IB, _IB),
                       edge_index[1].reshape(ep // _IB, _IB)), axis=1)

    [e1] = _edge_embed(edge_attr, [params[0]["We"]], [params[0]["be"]])
    batch3d = batch.reshape(n // 400, 1, 400)

    aggr2 = _make_mp(n, ep, 128, 128)(x, e1, idx3d)
    e2, e3 = _edge_embed(edge_attr, [p["We"] for p in params[1:]],
                         [p["be"] for p in params[1:]])
    h = _node_update(x, aggr2, w1p[0], b1s[0], fw2[0], fb2[0])
    for li, ee in ((1, e2), (2, e3)):
        mp = _make_mp(n, ep, 128, params[li]["We"].shape[1])
        aggr2 = mp(h, ee, idx3d)
        if li < 2:
            h = _node_update(h, aggr2, w1p[li], b1s[li], fw2[li], fb2[li])
        else:
            out = _node_update_pool(h, aggr2, batch3d, num_graphs,
                                    w1p[li], b1s[li], fw2[li], fb2[li])
    return out


# speedup vs baseline: 1.7178x; 1.1229x over previous
import functools

import jax
import jax.numpy as jnp
from jax import lax
from jax.experimental import pallas as pl
from jax.experimental.pallas import tpu as pltpu
from jax.experimental.pallas import tpu_sc as plsc

_HI = lax.Precision.HIGHEST


def _edge_embed_body(nl, ea_ref, *refs):
    ea = ea_ref[...]
    for j in range(nl):
        w, b = refs[2 * j], refs[2 * j + 1]
        refs[2 * nl + j][...] = jnp.dot(
            ea, w[...], preferred_element_type=jnp.float32,
            precision=_HI) + b[...]


def _edge_embed(edge_attr, ws, bs):
    e_num, d_e = edge_attr.shape
    dins = [w.shape[1] for w in ws]
    nl = len(ws)
    be = 2000
    grid = e_num // be
    full = lambda i: (0, 0)
    wspecs = []
    args = []
    for w, b in zip(ws, bs):
        wspecs += [pl.BlockSpec((d_e, w.shape[1]), full),
                   pl.BlockSpec((1, w.shape[1]), full)]
        args += [w, b[None, :]]
    out = pl.pallas_call(
        functools.partial(_edge_embed_body, nl),
        grid=(grid,),
        in_specs=[pl.BlockSpec((be, d_e), lambda i: (i, 0))] + wspecs,
        out_specs=[pl.BlockSpec((be, d), lambda i: (i, 0)) for d in dins],
        out_shape=[jax.ShapeDtypeStruct((e_num, d), jnp.float32) for d in dins],
    )(edge_attr, *args)
    return out


_IB = 128
_GRP = 16
_ZC = 80
_NPAD = 16


def _zero_rows(buf, rows, din):
    def _zb(i, _):
        for s in range(din // 16):
            buf[i, pl.ds(s * 16, 16)] = jnp.zeros((16,), jnp.float32)
        return 0
    lax.fori_loop(0, rows, _zb, 0)


def _chunk_loop(sid, ns, n, body):
    nz_tot = n // _ZC
    nch = jnp.where(sid < (nz_tot % ns), nz_tot // ns + 1, nz_tot // ns)

    def _it(j, _):
        off = pl.multiple_of((j * ns + sid) * _ZC, 8)
        body(off)
        return 0
    lax.fori_loop(0, nch, _it, 0)


def _relu_add(hbuf, ebuf, din):
    def _ew(i, _):
        for r in range(2):
            ii = 2 * i + r
            for s in range(din // 16):
                sl = pl.ds(s * 16, 16)
                hbuf[ii, sl] = jnp.maximum(hbuf[ii, sl] + ebuf[ii, sl], 0.0)
        return 0
    lax.fori_loop(0, _IB // 2, _ew, 0)


@functools.cache
def _make_mp(n, ep, din, de=128):
    info = plsc.get_sparse_core_info()
    nc, ns = info.num_cores, info.num_subcores
    nw = nc * ns
    nb = ep // _IB

    mesh = plsc.VectorSubcoreMesh(core_axis_name="c", subcore_axis_name="s")

    @functools.partial(
        pl.kernel,
        out_type=jax.ShapeDtypeStruct((nc, n, din), jnp.float32),
        mesh=mesh,
        scratch_types=[
            pltpu.VMEM_SHARED((n, din), jnp.float32),
            pltpu.VMEM((2, _IB), jnp.int32),
            pltpu.VMEM((_IB, din), jnp.float32),
            pltpu.VMEM((_IB, de), jnp.float32),
            pltpu.SemaphoreType.DMA,
            pltpu.SemaphoreType.DMA,
        ],
    )
    def mp(h_hbm, ee_hbm, idx_hbm, out_hbm, aggr, idxb, hbuf, ebuf, gs, es):
        cid = lax.axis_index("c")
        sid = lax.axis_index("s")
        wid = cid * ns + sid
        lo = (wid * nb) // nw
        hi = ((wid + 1) * nb) // nw

        _zero_rows(hbuf, _ZC, din)
        _chunk_loop(sid, ns, n, lambda off: pltpu.sync_copy(
            hbuf.at[pl.ds(0, _ZC)], aggr.at[pl.ds(off, _ZC)]))

        plsc.subcore_barrier()

        def _batch(b, _):
            eb = pl.multiple_of(b * _IB, _IB)
            pltpu.sync_copy(idx_hbm.at[b], idxb)
            pltpu.async_copy(ee_hbm.at[pl.ds(eb, _IB)], ebuf, es)
            pltpu.async_copy(h_hbm.at[idxb.at[0]], hbuf, gs).wait()
            pltpu.make_async_copy(ee_hbm.at[pl.ds(0, _IB)], ebuf, es).wait()
            _relu_add(hbuf, ebuf, de)
            pltpu.sync_copy(hbuf, aggr.at[idxb.at[1]], add=True)
            return 0
        lax.fori_loop(lo, hi, _batch, 0)

        plsc.subcore_barrier()
        _chunk_loop(sid, ns, n, lambda off: pltpu.sync_copy(
            aggr.at[pl.ds(off, _ZC)], out_hbm.at[cid, pl.ds(off, _ZC)]))

    return mp


def _node_body(h_ref, a_ref, w1, b1, w2, b2, o_ref):
    z = h_ref[...] + a_ref[0] + a_ref[1]
    t = jnp.maximum(jnp.dot(z, w1[...], preferred_element_type=jnp.float32,
                            precision=_HI) + b1[...], 0.0)
    o_ref[...] = jnp.maximum(jnp.dot(t, w2[...], preferred_element_type=jnp.float32,
                                     precision=_HI) + b2[...], 0.0)


def _node_update(h, aggr2, w1, b1, w2, b2, bn_rows=400):
    n, din = h.shape
    dm = w1.shape[1]
    dout = w2.shape[1]
    grid = n // bn_rows
    full = lambda i: (0, 0)
    return pl.pallas_call(
        _node_body,
        grid=(grid,),
        in_specs=[
            pl.BlockSpec((bn_rows, din), lambda i: (i, 0)),
            pl.BlockSpec((2, bn_rows, din), lambda i: (0, i, 0)),
            pl.BlockSpec((din, dm), full), pl.BlockSpec((1, dm), full),
            pl.BlockSpec((dm, dout), full), pl.BlockSpec((1, dout), full),
        ],
        out_specs=pl.BlockSpec((bn_rows, dout), lambda i: (i, 0)),
        out_shape=jax.ShapeDtypeStruct((n, dout), jnp.float32),
    )(h, aggr2, w1, b1[None, :], w2, b2[None, :])


def _node_pool_body(ng, h_ref, a_ref, batch_ref, w1, b1, w2, b2, o_ref, cnt):
    i = pl.program_id(0)

    @pl.when(i == 0)
    def _():
        o_ref[...] = jnp.zeros_like(o_ref)
        cnt[...] = jnp.zeros_like(cnt)

    z = h_ref[...] + a_ref[0] + a_ref[1]
    t = jnp.maximum(jnp.dot(z, w1[...], preferred_element_type=jnp.float32,
                            precision=_HI) + b1[...], 0.0)
    h3 = jnp.maximum(jnp.dot(t, w2[...], preferred_element_type=jnp.float32,
                             precision=_HI) + b2[...], 0.0)
    g = o_ref.shape[0]
    gids = lax.broadcasted_iota(jnp.int32, (g, h3.shape[0]), 0)
    onehot = (gids == batch_ref[0]).astype(jnp.float32)
    o_ref[...] += jnp.dot(onehot, h3, preferred_element_type=jnp.float32,
                          precision=_HI)
    cnt[...] += jnp.sum(onehot, axis=1, keepdims=True)

    @pl.when(i == ng - 1)
    def _():
        o_ref[...] = o_ref[...] / jnp.maximum(cnt[:, :1], 1.0)


def _node_update_pool(h, aggr2, batch3d, num_graphs, w1, b1, w2, b2, bn_rows=400):
    n, din = h.shape
    dm = w1.shape[1]
    dout = w2.shape[1]
    grid = n // bn_rows
    full = lambda i: (0, 0)
    return pl.pallas_call(
        functools.partial(_node_pool_body, grid),
        grid=(grid,),
        in_specs=[
            pl.BlockSpec((bn_rows, din), lambda i: (i, 0)),
            pl.BlockSpec((2, bn_rows, din), lambda i: (0, i, 0)),
            pl.BlockSpec((1, 1, bn_rows), lambda i: (i, 0, 0)),
            pl.BlockSpec((din, dm), full), pl.BlockSpec((1, dm), full),
            pl.BlockSpec((dm, dout), full), pl.BlockSpec((1, dout), full),
        ],
        out_specs=pl.BlockSpec((num_graphs, dout), lambda i: (0, 0)),
        out_shape=jax.ShapeDtypeStruct((num_graphs, dout), jnp.float32),
        scratch_shapes=[pltpu.VMEM((num_graphs, 128), jnp.float32)],
        compiler_params=pltpu.CompilerParams(
            dimension_semantics=("arbitrary",)),
    )(h, aggr2, batch3d, w1, b1[None, :], w2, b2[None, :])


def kernel(x, edge_index, edge_attr, batch, params, bn_stats):
    n, _ = x.shape
    e_num = edge_attr.shape[0]
    num_graphs = 64
    eps_bn = 1e-5

    dpad = 128
    w1p, b1s, fw2, fb2 = [], [], [], []
    for li, (p, st) in enumerate(zip(params, bn_stats)):
        din, dm = p["W1"].shape
        dout = p["W2"].shape[1]
        scale = p["gamma"] / jnp.sqrt(st["var"] + eps_bn)
        w2f = p["W2"] * scale[None, :]
        b2f = (p["b2"] - st["mean"]) * scale + p["beta"]
        w1p.append(jnp.pad(p["W1"], ((0, dpad - din), (0, 0))))
        b1s.append(p["b1"])
        if li < 2:
            w2f = jnp.pad(w2f, ((0, 0), (0, dpad - dout)))
            b2f = jnp.pad(b2f, (0, dpad - dout))
        fw2.append(w2f)
        fb2.append(b2f)

    ep = e_num
    idx3d = jnp.stack((edge_index[0].reshape(ep // _IB, _IB),
                       edge_index[1].reshape(ep // _IB, _IB)), axis=1)

    [e1] = _edge_embed(edge_attr, [params[0]["We"]], [params[0]["be"]])
    batch3d = batch.reshape(n // 400, 1, 400)

    aggr2 = _make_mp(n, ep, 128, 128)(x, e1, idx3d)
    e2, e3 = _edge_embed(edge_attr, [p["We"] for p in params[1:]],
                         [p["be"] for p in params[1:]])
    h = _node_update(x, aggr2, w1p[0], b1s[0], fw2[0], fb2[0])
    for li, ee in ((1, e2), (2, e3)):
        mp = _make_mp(n, ep, 128, params[li]["We"].shape[1])
        aggr2 = mp(h, ee, idx3d)
        if li < 2:
            h = _node_update(h, aggr2, w1p[li], b1s[li], fw2[li], fb2[li])
        else:
            out = _node_update_pool(h, aggr2, batch3d, num_graphs,
                                    w1p[li], b1s[li], fw2[li], fb2[li])
    return out


# speedup vs baseline: 1.8810x; 1.0950x over previous
import functools

import jax
import jax.numpy as jnp
from jax import lax
from jax.experimental import pallas as pl
from jax.experimental.pallas import tpu as pltpu
from jax.experimental.pallas import tpu_sc as plsc

_HI = lax.Precision.HIGHEST


def _edge_embed_body(nl, ea_ref, *refs):
    ea = ea_ref[...]
    for j in range(nl):
        w, b = refs[2 * j], refs[2 * j + 1]
        refs[2 * nl + j][...] = jnp.dot(
            ea, w[...], preferred_element_type=jnp.float32,
            precision=_HI) + b[...]


def _edge_embed(edge_attr, ws, bs):
    e_num, d_e = edge_attr.shape
    dins = [w.shape[1] for w in ws]
    nl = len(ws)
    be = 2000
    grid = e_num // be
    full = lambda i: (0, 0)
    wspecs = []
    args = []
    for w, b in zip(ws, bs):
        wspecs += [pl.BlockSpec((d_e, w.shape[1]), full),
                   pl.BlockSpec((1, w.shape[1]), full)]
        args += [w, b[None, :]]
    out = pl.pallas_call(
        functools.partial(_edge_embed_body, nl),
        grid=(grid,),
        in_specs=[pl.BlockSpec((be, d_e), lambda i: (i, 0))] + wspecs,
        out_specs=[pl.BlockSpec((be, d), lambda i: (i, 0)) for d in dins],
        out_shape=[jax.ShapeDtypeStruct((e_num, d), jnp.float32) for d in dins],
    )(edge_attr, *args)
    return out


_IB = 128
_GRP = 16
_ZC = 80
_NPAD = 16


def _zero_rows(buf, rows, din):
    def _zb(i, _):
        for s in range(din // 16):
            buf[i, pl.ds(s * 16, 16)] = jnp.zeros((16,), jnp.float32)
        return 0
    lax.fori_loop(0, rows, _zb, 0)


def _chunk_loop(sid, ns, n, body):
    nz_tot = n // _ZC
    nch = jnp.where(sid < (nz_tot % ns), nz_tot // ns + 1, nz_tot // ns)

    def _it(j, _):
        off = pl.multiple_of((j * ns + sid) * _ZC, 8)
        body(off)
        return 0
    lax.fori_loop(0, nch, _it, 0)


def _relu_add(hbuf, ebuf, din):
    def _ew(i, _):
        for r in range(2):
            ii = 2 * i + r
            for s in range(din // 16):
                sl = pl.ds(s * 16, 16)
                hbuf[ii, sl] = jnp.maximum(hbuf[ii, sl] + ebuf[ii, sl], 0.0)
        return 0
    lax.fori_loop(0, _IB // 2, _ew, 0)


@functools.cache
def _make_mp(n, ep, din, de=128):
    info = plsc.get_sparse_core_info()
    nc, ns = info.num_cores, info.num_subcores
    nw = nc * ns
    nb = ep // _IB

    mesh = plsc.VectorSubcoreMesh(core_axis_name="c", subcore_axis_name="s")

    @functools.partial(
        pl.kernel,
        out_type=jax.ShapeDtypeStruct((nc, n, din), jnp.float32),
        mesh=mesh,
        scratch_types=[
            pltpu.VMEM_SHARED((n, din), jnp.float32),
            pltpu.VMEM((2, _IB), jnp.int32),
            pltpu.VMEM((_IB, din), jnp.float32),
            pltpu.VMEM((_IB, de), jnp.float32),
            pltpu.SemaphoreType.DMA,
            pltpu.SemaphoreType.DMA,
            pltpu.SemaphoreType.DMA,
        ],
    )
    def mp(h_hbm, ee_hbm, idx_hbm, out_hbm, aggr, idxb, hbuf, ebuf, gs, es, ss):
        cid = lax.axis_index("c")
        sid = lax.axis_index("s")
        wid = cid * ns + sid
        lo = (wid * nb) // nw
        hi = ((wid + 1) * nb) // nw

        _zero_rows(hbuf, _ZC, din)
        _chunk_loop(sid, ns, n, lambda off: pltpu.sync_copy(
            hbuf.at[pl.ds(0, _ZC)], aggr.at[pl.ds(off, _ZC)]))

        plsc.subcore_barrier()

        def _batch(b, _):
            eb = pl.multiple_of(b * _IB, _IB)
            pltpu.sync_copy(idx_hbm.at[b], idxb)
            pltpu.async_copy(ee_hbm.at[pl.ds(eb, _IB)], ebuf, es)

            @pl.when(b > lo)
            def _():
                pltpu.make_async_copy(hbuf, aggr.at[idxb.at[1]], ss).wait()
            pltpu.async_copy(h_hbm.at[idxb.at[0]], hbuf, gs).wait()
            pltpu.make_async_copy(ee_hbm.at[pl.ds(0, _IB)], ebuf, es).wait()
            _relu_add(hbuf, ebuf, de)
            pltpu.async_copy(hbuf, aggr.at[idxb.at[1]], ss, add=True)
            return 0
        lax.fori_loop(lo, hi, _batch, 0)

        @pl.when(hi > lo)
        def _():
            pltpu.make_async_copy(hbuf, aggr.at[idxb.at[1]], ss).wait()
        plsc.subcore_barrier()
        _chunk_loop(sid, ns, n, lambda off: pltpu.sync_copy(
            aggr.at[pl.ds(off, _ZC)], out_hbm.at[cid, pl.ds(off, _ZC)]))

    return mp


def _node_body(h_ref, a_ref, w1, b1, w2, b2, o_ref):
    z = h_ref[...] + a_ref[0] + a_ref[1]
    t = jnp.maximum(jnp.dot(z, w1[...], preferred_element_type=jnp.float32,
                            precision=_HI) + b1[...], 0.0)
    o_ref[...] = jnp.maximum(jnp.dot(t, w2[...], preferred_element_type=jnp.float32,
                                     precision=_HI) + b2[...], 0.0)


def _node_update(h, aggr2, w1, b1, w2, b2, bn_rows=400):
    n, din = h.shape
    dm = w1.shape[1]
    dout = w2.shape[1]
    grid = n // bn_rows
    full = lambda i: (0, 0)
    return pl.pallas_call(
        _node_body,
        grid=(grid,),
        in_specs=[
            pl.BlockSpec((bn_rows, din), lambda i: (i, 0)),
            pl.BlockSpec((2, bn_rows, din), lambda i: (0, i, 0)),
            pl.BlockSpec((din, dm), full), pl.BlockSpec((1, dm), full),
            pl.BlockSpec((dm, dout), full), pl.BlockSpec((1, dout), full),
        ],
        out_specs=pl.BlockSpec((bn_rows, dout), lambda i: (i, 0)),
        out_shape=jax.ShapeDtypeStruct((n, dout), jnp.float32),
    )(h, aggr2, w1, b1[None, :], w2, b2[None, :])


def _node_pool_body(ng, h_ref, a_ref, batch_ref, w1, b1, w2, b2, o_ref, cnt):
    i = pl.program_id(0)

    @pl.when(i == 0)
    def _():
        o_ref[...] = jnp.zeros_like(o_ref)
        cnt[...] = jnp.zeros_like(cnt)

    z = h_ref[...] + a_ref[0] + a_ref[1]
    t = jnp.maximum(jnp.dot(z, w1[...], preferred_element_type=jnp.float32,
                            precision=_HI) + b1[...], 0.0)
    h3 = jnp.maximum(jnp.dot(t, w2[...], preferred_element_type=jnp.float32,
                             precision=_HI) + b2[...], 0.0)
    g = o_ref.shape[0]
    gids = lax.broadcasted_iota(jnp.int32, (g, h3.shape[0]), 0)
    onehot = (gids == batch_ref[0]).astype(jnp.float32)
    o_ref[...] += jnp.dot(onehot, h3, preferred_element_type=jnp.float32,
                          precision=_HI)
    cnt[...] += jnp.sum(onehot, axis=1, keepdims=True)

    @pl.when(i == ng - 1)
    def _():
        o_ref[...] = o_ref[...] / jnp.maximum(cnt[:, :1], 1.0)


def _node_update_pool(h, aggr2, batch3d, num_graphs, w1, b1, w2, b2, bn_rows=400):
    n, din = h.shape
    dm = w1.shape[1]
    dout = w2.shape[1]
    grid = n // bn_rows
    full = lambda i: (0, 0)
    return pl.pallas_call(
        functools.partial(_node_pool_body, grid),
        grid=(grid,),
        in_specs=[
            pl.BlockSpec((bn_rows, din), lambda i: (i, 0)),
            pl.BlockSpec((2, bn_rows, din), lambda i: (0, i, 0)),
            pl.BlockSpec((1, 1, bn_rows), lambda i: (i, 0, 0)),
            pl.BlockSpec((din, dm), full), pl.BlockSpec((1, dm), full),
            pl.BlockSpec((dm, dout), full), pl.BlockSpec((1, dout), full),
        ],
        out_specs=pl.BlockSpec((num_graphs, dout), lambda i: (0, 0)),
        out_shape=jax.ShapeDtypeStruct((num_graphs, dout), jnp.float32),
        scratch_shapes=[pltpu.VMEM((num_graphs, 128), jnp.float32)],
        compiler_params=pltpu.CompilerParams(
            dimension_semantics=("arbitrary",)),
    )(h, aggr2, batch3d, w1, b1[None, :], w2, b2[None, :])


def kernel(x, edge_index, edge_attr, batch, params, bn_stats):
    n, _ = x.shape
    e_num = edge_attr.shape[0]
    num_graphs = 64
    eps_bn = 1e-5

    dpad = 128
    w1p, b1s, fw2, fb2 = [], [], [], []
    for li, (p, st) in enumerate(zip(params, bn_stats)):
        din, dm = p["W1"].shape
        dout = p["W2"].shape[1]
        scale = p["gamma"] / jnp.sqrt(st["var"] + eps_bn)
        w2f = p["W2"] * scale[None, :]
        b2f = (p["b2"] - st["mean"]) * scale + p["beta"]
        w1p.append(jnp.pad(p["W1"], ((0, dpad - din), (0, 0))))
        b1s.append(p["b1"])
        if li < 2:
            w2f = jnp.pad(w2f, ((0, 0), (0, dpad - dout)))
            b2f = jnp.pad(b2f, (0, dpad - dout))
        fw2.append(w2f)
        fb2.append(b2f)

    ep = e_num
    idx3d = jnp.stack((edge_index[0].reshape(ep // _IB, _IB),
                       edge_index[1].reshape(ep // _IB, _IB)), axis=1)

    [e1] = _edge_embed(edge_attr, [params[0]["We"]], [params[0]["be"]])
    batch3d = batch.reshape(n // 400, 1, 400)

    aggr2 = _make_mp(n, ep, 128, 128)(x, e1, idx3d)
    e2, e3 = _edge_embed(edge_attr, [p["We"] for p in params[1:]],
                         [p["be"] for p in params[1:]])
    h = _node_update(x, aggr2, w1p[0], b1s[0], fw2[0], fb2[0])
    for li, ee in ((1, e2), (2, e3)):
        mp = _make_mp(n, ep, 128, params[li]["We"].shape[1])
        aggr2 = mp(h, ee, idx3d)
        if li < 2:
            h = _node_update(h, aggr2, w1p[li], b1s[li], fw2[li], fb2[li])
        else:
            out = _node_update_pool(h, aggr2, batch3d, num_graphs,
                                    w1p[li], b1s[li], fw2[li], fb2[li])
    return out


# speedup vs baseline: 1.8817x; 1.0004x over previous
import functools

import jax
import jax.numpy as jnp
from jax import lax
from jax.experimental import pallas as pl
from jax.experimental.pallas import tpu as pltpu
from jax.experimental.pallas import tpu_sc as plsc

_HI = lax.Precision.HIGHEST


def _edge_embed_body(nl, ea_ref, *refs):
    ea = ea_ref[...]
    for j in range(nl):
        w, b = refs[2 * j], refs[2 * j + 1]
        refs[2 * nl + j][...] = jnp.dot(
            ea, w[...], preferred_element_type=jnp.float32,
            precision=_HI) + b[...]


def _edge_embed(edge_attr, ws, bs):
    e_num, d_e = edge_attr.shape
    dins = [w.shape[1] for w in ws]
    nl = len(ws)
    be = 2000
    grid = e_num // be
    full = lambda i: (0, 0)
    wspecs = []
    args = []
    for w, b in zip(ws, bs):
        wspecs += [pl.BlockSpec((d_e, w.shape[1]), full),
                   pl.BlockSpec((1, w.shape[1]), full)]
        args += [w, b[None, :]]
    out = pl.pallas_call(
        functools.partial(_edge_embed_body, nl),
        grid=(grid,),
        in_specs=[pl.BlockSpec((be, d_e), lambda i: (i, 0))] + wspecs,
        out_specs=[pl.BlockSpec((be, d), lambda i: (i, 0)) for d in dins],
        out_shape=[jax.ShapeDtypeStruct((e_num, d), jnp.float32) for d in dins],
    )(edge_attr, *args)
    return out


_IB = 128
_ZC = 80


def _zero_rows(buf, rows, din):
    def _zb(i, _):
        for s in range(din // 16):
            buf[i, pl.ds(s * 16, 16)] = jnp.zeros((16,), jnp.float32)
        return 0
    lax.fori_loop(0, rows, _zb, 0)


def _chunk_loop(sid, ns, n, body):
    nz_tot = n // _ZC
    nch = jnp.where(sid < (nz_tot % ns), nz_tot // ns + 1, nz_tot // ns)

    def _it(j, _):
        off = pl.multiple_of((j * ns + sid) * _ZC, 8)
        body(off)
        return 0
    lax.fori_loop(0, nch, _it, 0)


def _relu_add(hbuf, ebuf, din):
    def _ew(i, _):
        for r in range(2):
            ii = 2 * i + r
            for s in range(din // 16):
                sl = pl.ds(s * 16, 16)
                hbuf[ii, sl] = jnp.maximum(hbuf[ii, sl] + ebuf[ii, sl], 0.0)
        return 0
    lax.fori_loop(0, _IB // 2, _ew, 0)


@functools.cache
def _make_mp(n, ep, din, de=128):
    info = plsc.get_sparse_core_info()
    nc, ns = info.num_cores, info.num_subcores
    nw = nc * ns
    nb = ep // _IB

    mesh = plsc.VectorSubcoreMesh(core_axis_name="c", subcore_axis_name="s")

    @functools.partial(
        pl.kernel,
        out_type=jax.ShapeDtypeStruct((nc, n, din), jnp.float32),
        mesh=mesh,
        scratch_types=[
            pltpu.VMEM_SHARED((n, din), jnp.float32),
            pltpu.VMEM((2, _IB), jnp.int32),
            pltpu.VMEM((_IB, din), jnp.float32),
            pltpu.VMEM((_IB, de), jnp.float32),
            pltpu.SemaphoreType.DMA,
            pltpu.SemaphoreType.DMA,
            pltpu.SemaphoreType.DMA,
        ],
    )
    def mp(h_hbm, ee_hbm, idx_hbm, out_hbm, aggr, idxb, hbuf, ebuf, gs, es, ss):
        cid = lax.axis_index("c")
        sid = lax.axis_index("s")
        wid = cid * ns + sid
        lo = (wid * nb) // nw
        hi = ((wid + 1) * nb) // nw

        _zero_rows(hbuf, _ZC, din)
        _chunk_loop(sid, ns, n, lambda off: pltpu.sync_copy(
            hbuf.at[pl.ds(0, _ZC)], aggr.at[pl.ds(off, _ZC)]))

        plsc.subcore_barrier()

        def _batch(b, _):
            eb = pl.multiple_of(b * _IB, _IB)
            pltpu.sync_copy(idx_hbm.at[b], idxb)
            pltpu.async_copy(ee_hbm.at[pl.ds(eb, _IB)], ebuf, es)

            @pl.when(b > lo)
            def _():
                pltpu.make_async_copy(hbuf, aggr.at[idxb.at[1]], ss).wait()
            pltpu.async_copy(h_hbm.at[idxb.at[0]], hbuf, gs).wait()
            pltpu.make_async_copy(ee_hbm.at[pl.ds(0, _IB)], ebuf, es).wait()
            _relu_add(hbuf, ebuf, de)
            pltpu.async_copy(hbuf, aggr.at[idxb.at[1]], ss, add=True)
            return 0
        lax.fori_loop(lo, hi, _batch, 0)

        @pl.when(hi > lo)
        def _():
            pltpu.make_async_copy(hbuf, aggr.at[idxb.at[1]], ss).wait()
        plsc.subcore_barrier()
        _chunk_loop(sid, ns, n, lambda off: pltpu.sync_copy(
            aggr.at[pl.ds(off, _ZC)], out_hbm.at[cid, pl.ds(off, _ZC)]))

    return mp


def _node_body(h_ref, a_ref, w1, b1, w2, b2, o_ref):
    z = h_ref[...] + a_ref[0] + a_ref[1]
    t = jnp.maximum(jnp.dot(z, w1[...], preferred_element_type=jnp.float32,
                            precision=_HI) + b1[...], 0.0)
    o_ref[...] = jnp.maximum(jnp.dot(t, w2[...], preferred_element_type=jnp.float32,
                                     precision=_HI) + b2[...], 0.0)


def _node_update(h, aggr2, w1, b1, w2, b2, bn_rows=400):
    n, din = h.shape
    dm = w1.shape[1]
    dout = w2.shape[1]
    grid = n // bn_rows
    full = lambda i: (0, 0)
    return pl.pallas_call(
        _node_body,
        grid=(grid,),
        in_specs=[
            pl.BlockSpec((bn_rows, din), lambda i: (i, 0)),
            pl.BlockSpec((2, bn_rows, din), lambda i: (0, i, 0)),
            pl.BlockSpec((din, dm), full), pl.BlockSpec((1, dm), full),
            pl.BlockSpec((dm, dout), full), pl.BlockSpec((1, dout), full),
        ],
        out_specs=pl.BlockSpec((bn_rows, dout), lambda i: (i, 0)),
        out_shape=jax.ShapeDtypeStruct((n, dout), jnp.float32),
    )(h, aggr2, w1, b1[None, :], w2, b2[None, :])


def _node_pool_body(ng, h_ref, a_ref, batch_ref, w1, b1, w2, b2, o_ref, cnt):
    i = pl.program_id(0)

    @pl.when(i == 0)
    def _():
        o_ref[...] = jnp.zeros_like(o_ref)
        cnt[...] = jnp.zeros_like(cnt)

    z = h_ref[...] + a_ref[0] + a_ref[1]
    t = jnp.maximum(jnp.dot(z, w1[...], preferred_element_type=jnp.float32,
                            precision=_HI) + b1[...], 0.0)
    h3 = jnp.maximum(jnp.dot(t, w2[...], preferred_element_type=jnp.float32,
                             precision=_HI) + b2[...], 0.0)
    g = o_ref.shape[0]
    gids = lax.broadcasted_iota(jnp.int32, (g, h3.shape[0]), 0)
    onehot = (gids == batch_ref[0]).astype(jnp.float32)
    o_ref[...] += jnp.dot(onehot, h3, preferred_element_type=jnp.float32,
                          precision=_HI)
    cnt[...] += jnp.sum(onehot, axis=1, keepdims=True)

    @pl.when(i == ng - 1)
    def _():
        o_ref[...] = o_ref[...] / jnp.maximum(cnt[:, :1], 1.0)


def _node_update_pool(h, aggr2, batch3d, num_graphs, w1, b1, w2, b2, bn_rows=400):
    n, din = h.shape
    dm = w1.shape[1]
    dout = w2.shape[1]
    grid = n // bn_rows
    full = lambda i: (0, 0)
    return pl.pallas_call(
        functools.partial(_node_pool_body, grid),
        grid=(grid,),
        in_specs=[
            pl.BlockSpec((bn_rows, din), lambda i: (i, 0)),
            pl.BlockSpec((2, bn_rows, din), lambda i: (0, i, 0)),
            pl.BlockSpec((1, 1, bn_rows), lambda i: (i, 0, 0)),
            pl.BlockSpec((din, dm), full), pl.BlockSpec((1, dm), full),
            pl.BlockSpec((dm, dout), full), pl.BlockSpec((1, dout), full),
        ],
        out_specs=pl.BlockSpec((num_graphs, dout), lambda i: (0, 0)),
        out_shape=jax.ShapeDtypeStruct((num_graphs, dout), jnp.float32),
        scratch_shapes=[pltpu.VMEM((num_graphs, 128), jnp.float32)],
        compiler_params=pltpu.CompilerParams(
            dimension_semantics=("arbitrary",)),
    )(h, aggr2, batch3d, w1, b1[None, :], w2, b2[None, :])


def kernel(x, edge_index, edge_attr, batch, params, bn_stats):
    n, _ = x.shape
    e_num = edge_attr.shape[0]
    num_graphs = 64
    eps_bn = 1e-5

    dpad = 128
    w1p, b1s, fw2, fb2 = [], [], [], []
    for li, (p, st) in enumerate(zip(params, bn_stats)):
        din, dm = p["W1"].shape
        dout = p["W2"].shape[1]
        scale = p["gamma"] / jnp.sqrt(st["var"] + eps_bn)
        w2f = p["W2"] * scale[None, :]
        b2f = (p["b2"] - st["mean"]) * scale + p["beta"]
        w1p.append(jnp.pad(p["W1"], ((0, dpad - din), (0, 0))))
        b1s.append(p["b1"])
        if li < 2:
            w2f = jnp.pad(w2f, ((0, 0), (0, dpad - dout)))
            b2f = jnp.pad(b2f, (0, dpad - dout))
        fw2.append(w2f)
        fb2.append(b2f)

    ep = e_num
    idx3d = jnp.stack((edge_index[0].reshape(ep // _IB, _IB),
                       edge_index[1].reshape(ep // _IB, _IB)), axis=1)

    [e1] = _edge_embed(edge_attr, [params[0]["We"]], [params[0]["be"]])
    batch3d = batch.reshape(n // 400, 1, 400)

    aggr2 = _make_mp(n, ep, 128, 128)(x, e1, idx3d)
    e2, e3 = _edge_embed(edge_attr, [p["We"] for p in params[1:]],
                         [p["be"] for p in params[1:]])
    h = _node_update(x, aggr2, w1p[0], b1s[0], fw2[0], fb2[0])
    for li, ee in ((1, e2), (2, e3)):
        mp = _make_mp(n, ep, 128, params[li]["We"].shape[1])
        aggr2 = mp(h, ee, idx3d)
        if li < 2:
            h = _node_update(h, aggr2, w1p[li], b1s[li], fw2[li], fb2[li])
        else:
            out = _node_update_pool(h, aggr2, batch3d, num_graphs,
                                    w1p[li], b1s[li], fw2[li], fb2[li])
    return out
